# baseline (speedup 1.0000x reference)
import functools

import jax
import jax.numpy as jnp
from jax import lax
from jax.experimental import pallas as pl
from jax.experimental.pallas import tpu as pltpu
from jax.experimental.pallas import tpu_sc as plsc

NUM_NODES = 100
BATCH = 100
N = NUM_NODES * BATCH
E_NUM = 320000
NUM_OBJ_CLS = 100
NC, NS = 2, 16
NW = NC * NS
SR = 512
NSUP = E_NUM // SR
SPW = NSUP // NW
XSUP = NSUP - SPW * NW
NSLOT = 2
RPT = 624
RPT_LAST = N - 15 * RPT


@functools.cache
def _make_scatter(D):
    mesh = plsc.VectorSubcoreMesh(core_axis_name="c", subcore_axis_name="s",
                                  num_cores=NC, num_subcores=NS)

    @functools.partial(
        pl.kernel,
        out_type=jax.ShapeDtypeStruct((NC, N, D), jnp.float32),
        mesh=mesh,
        scratch_types=[
            pltpu.VMEM((SPW + 1, SR), jnp.int32),
            pltpu.VMEM((SPW + 1, SR), jnp.int32),
            pltpu.VMEM((NSLOT, SR, D), jnp.float32),
            pltpu.VMEM_SHARED((N, D), jnp.float32),
            pltpu.SemaphoreType.DMA((NSLOT,)),
            pltpu.SemaphoreType.DMA((NSLOT,)),
        ],
        compiler_params=pltpu.CompilerParams(use_tc_tiling_on_sc=False),
    )
    def scatter(e_hbm, h_hbm, zeros_hbm, out_hbm,
                src_v, dst_v, rows_v, acc, gsem, ssem):
        c = lax.axis_index("c")
        s = lax.axis_index("s")
        wid = s * NC + c
        r0 = pl.multiple_of(s * RPT, 8)

        row0 = wid * SPW
        pltpu.sync_copy(e_hbm.at[0, pl.ds(row0, SPW)], src_v.at[pl.ds(0, SPW)])
        pltpu.sync_copy(e_hbm.at[1, pl.ds(row0, SPW)], dst_v.at[pl.ds(0, SPW)])

        @pl.when(wid < XSUP)
        def _():
            xr = NW * SPW + wid
            pltpu.sync_copy(e_hbm.at[0, pl.ds(xr, 1)], src_v.at[pl.ds(SPW, 1)])
            pltpu.sync_copy(e_hbm.at[1, pl.ds(xr, 1)], dst_v.at[pl.ds(SPW, 1)])

        @pl.when(s < NS - 1)
        def _():
            pltpu.sync_copy(zeros_hbm.at[pl.ds(r0, RPT)],
                            acc.at[pl.ds(r0, RPT)])

        @pl.when(s == NS - 1)
        def _():
            pltpu.sync_copy(zeros_hbm.at[pl.ds(15 * RPT, RPT_LAST)],
                            acc.at[pl.ds(15 * RPT, RPT_LAST)])

        plsc.subcore_barrier()

        def do_group(k, b):
            pltpu.make_async_copy(h_hbm.at[src_v.at[k]],
                                  rows_v.at[b], gsem.at[b]).wait()
            pltpu.async_copy(rows_v.at[b], acc.at[dst_v.at[k]],
                             ssem.at[b], add=True)
            pltpu.make_async_copy(rows_v.at[b], acc.at[dst_v.at[k]],
                                  ssem.at[b]).wait()

            @pl.when(k + NSLOT < SPW)
            def _():
                pltpu.async_copy(h_hbm.at[src_v.at[k + NSLOT]],
                                 rows_v.at[b], gsem.at[b])

        for b in range(NSLOT):
            pltpu.async_copy(h_hbm.at[src_v.at[b]], rows_v.at[b],
                             gsem.at[b])

        def body(g2, carry):
            for b in range(NSLOT):
                do_group(g2 * NSLOT + b, b)
            return carry

        lax.fori_loop(0, SPW // NSLOT, body, 0)
        for k in range(SPW - SPW % NSLOT, SPW):
            do_group(k, k % NSLOT)

        @pl.when(wid < XSUP)
        def _():
            pltpu.async_copy(h_hbm.at[src_v.at[SPW]],
                             rows_v.at[0], gsem.at[0]).wait()
            pltpu.async_copy(rows_v.at[0],
                             acc.at[dst_v.at[SPW]],
                             ssem.at[0], add=True).wait()

        plsc.subcore_barrier()

        @pl.when(s < NS - 1)
        def _():
            pltpu.sync_copy(acc.at[pl.ds(r0, RPT)],
                            out_hbm.at[c, pl.ds(r0, RPT)])

        @pl.when(s == NS - 1)
        def _():
            pltpu.sync_copy(acc.at[pl.ds(15 * RPT, RPT_LAST)],
                            out_hbm.at[c, pl.ds(15 * RPT, RPT_LAST)])

    return scatter


def _mm1_body(x_ref, w_ref, wb_ref, bb_ref, wl_ref, bl_ref,
              h_ref, mix_ref):
    relu = lambda v: jnp.maximum(v, 0.0)
    xd = x_ref[...]
    h_ref[...] = jnp.dot(xd, w_ref[...], preferred_element_type=jnp.float32)
    boxes = relu(jnp.dot(xd[:, 1:], wb_ref[...],
                         preferred_element_type=jnp.float32) + bb_ref[...])
    labels = relu(xd[:, 0:1] * wl_ref[...] + bl_ref[...])
    mix_ref[...] = boxes + labels


def _mm2_body(p_ref, w_ref, o_ref):
    x = p_ref[0] + p_ref[1]
    o_ref[...] = jnp.dot(x, w_ref[...], preferred_element_type=jnp.float32)


def _head_body(x0_ref, x1_ref, mixr_ref, cl_ref, wd1_ref, bd1_ref,
               wd2_ref, bd2_ref, wd3_ref, bd3_ref,
               wlat_ref, blat_ref, o_ref):
    relu = lambda v: jnp.maximum(v, 0.0)
    f32 = jnp.float32
    xr = x0_ref[...] + x1_ref[...]
    mix = relu(jnp.dot(mixr_ref[...], wd1_ref[...],
                       preferred_element_type=f32) + bd1_ref[...])
    x = relu(jnp.dot(cl_ref[...], wd2_ref[0:NUM_OBJ_CLS],
                     preferred_element_type=f32)
             + jnp.dot(xr, wd2_ref[NUM_OBJ_CLS:],
                       preferred_element_type=f32)
             + bd2_ref[...])
    x = x + mix
    x = relu(jnp.dot(x, wd3_ref[...], preferred_element_type=f32)
             + bd3_ref[...])
    x = relu(jnp.dot(x, wd3_ref[...], preferred_element_type=f32)
             + bd3_ref[...])
    o_ref[...] = relu(jnp.dot(x, wlat_ref[...], preferred_element_type=f32)
                      + blat_ref[...])


def kernel(E, X_data, class_labels, W1, W2, Wb, bb, Wl, bl,
           Wd1, bd1, Wd2, bd2, Wd3, bd3, Wlat, blat):
    f32 = jnp.float32
    E3 = E.reshape(2, NSUP, SR)
    zeros64 = jnp.zeros((N, 64), f32)
    zeros16 = jnp.zeros((N, 16), f32)

    h1, mixn = pl.pallas_call(
        _mm1_body,
        out_shape=(jax.ShapeDtypeStruct((N, 64), f32),
                   jax.ShapeDtypeStruct((N, 16), f32)),
    )(X_data, W1, Wb, bb.reshape(1, 16), Wl, bl.reshape(1, 16))

    p1 = _make_scatter(64)(E3, h1, zeros64)

    h2 = pl.pallas_call(
        _mm2_body,
        out_shape=jax.ShapeDtypeStruct((N, 16), f32),
    )(p1, W2)

    p2 = _make_scatter(16)(E3, h2, zeros16)

    x0_r = p2[0].reshape(BATCH, NUM_NODES * 16)
    x1_r = p2[1].reshape(BATCH, NUM_NODES * 16)
    mix_r = mixn.reshape(BATCH, NUM_NODES * 16)
    cl = class_labels.reshape(BATCH, -1)

    z = pl.pallas_call(
        _head_body,
        out_shape=jax.ShapeDtypeStruct((BATCH, 128), f32),
    )(x0_r, x1_r, mix_r, cl, Wd1, bd1.reshape(1, -1),
      Wd2, bd2.reshape(1, -1),
      Wd3, bd3.reshape(1, -1), Wlat, blat.reshape(1, -1))

    return (z, z)


# speedup vs baseline: 1.0601x; 1.0601x over previous
import functools

import jax
import jax.numpy as jnp
from jax import lax
from jax.experimental import pallas as pl
from jax.experimental.pallas import tpu as pltpu
from jax.experimental.pallas import tpu_sc as plsc

NUM_NODES = 100
BATCH = 100
N = NUM_NODES * BATCH
E_NUM = 320000
NUM_OBJ_CLS = 100
NC, NS = 2, 16
NW = NC * NS
SR = 256
NSUP = E_NUM // SR
SPW = NSUP // NW
XSUP = NSUP - SPW * NW
NSLOT = 4
RPT = 624
RPT_LAST = N - 15 * RPT


@functools.cache
def _make_scatter(D):
    mesh = plsc.VectorSubcoreMesh(core_axis_name="c", subcore_axis_name="s",
                                  num_cores=NC, num_subcores=NS)

    @functools.partial(
        pl.kernel,
        out_type=jax.ShapeDtypeStruct((NC, N, D), jnp.float32),
        mesh=mesh,
        scratch_types=[
            pltpu.VMEM((SPW + 1, SR), jnp.int32),
            pltpu.VMEM((SPW + 1, SR), jnp.int32),
            pltpu.VMEM((NSLOT, SR, D), jnp.float32),
            pltpu.VMEM_SHARED((N, D), jnp.float32),
            pltpu.SemaphoreType.DMA((NSLOT,)),
            pltpu.SemaphoreType.DMA((NSLOT,)),
        ],
        compiler_params=pltpu.CompilerParams(use_tc_tiling_on_sc=False),
    )
    def scatter(e_hbm, h_hbm, zeros_hbm, out_hbm,
                src_v, dst_v, rows_v, acc, gsem, ssem):
        c = lax.axis_index("c")
        s = lax.axis_index("s")
        wid = s * NC + c
        r0 = pl.multiple_of(s * RPT, 8)

        row0 = wid * SPW
        pltpu.sync_copy(e_hbm.at[0, pl.ds(row0, SPW)], src_v.at[pl.ds(0, SPW)])
        pltpu.sync_copy(e_hbm.at[1, pl.ds(row0, SPW)], dst_v.at[pl.ds(0, SPW)])

        @pl.when(wid < XSUP)
        def _():
            xr = NW * SPW + wid
            pltpu.sync_copy(e_hbm.at[0, pl.ds(xr, 1)], src_v.at[pl.ds(SPW, 1)])
            pltpu.sync_copy(e_hbm.at[1, pl.ds(xr, 1)], dst_v.at[pl.ds(SPW, 1)])

        @pl.when(s < NS - 1)
        def _():
            pltpu.sync_copy(zeros_hbm.at[pl.ds(r0, RPT)],
                            acc.at[pl.ds(r0, RPT)])

        @pl.when(s == NS - 1)
        def _():
            pltpu.sync_copy(zeros_hbm.at[pl.ds(15 * RPT, RPT_LAST)],
                            acc.at[pl.ds(15 * RPT, RPT_LAST)])

        plsc.subcore_barrier()

        def do_group(k, b):
            pltpu.make_async_copy(h_hbm.at[src_v.at[k]],
                                  rows_v.at[b], gsem.at[b]).wait()
            pltpu.async_copy(rows_v.at[b], acc.at[dst_v.at[k]],
                             ssem.at[b], add=True)
            pltpu.make_async_copy(rows_v.at[b], acc.at[dst_v.at[k]],
                                  ssem.at[b]).wait()

            @pl.when(k + NSLOT < SPW)
            def _():
                pltpu.async_copy(h_hbm.at[src_v.at[k + NSLOT]],
                                 rows_v.at[b], gsem.at[b])

        for b in range(NSLOT):
            pltpu.async_copy(h_hbm.at[src_v.at[b]], rows_v.at[b],
                             gsem.at[b])

        def body(g2, carry):
            for b in range(NSLOT):
                do_group(g2 * NSLOT + b, b)
            return carry

        lax.fori_loop(0, SPW // NSLOT, body, 0)
        for k in range(SPW - SPW % NSLOT, SPW):
            do_group(k, k % NSLOT)

        @pl.when(wid < XSUP)
        def _():
            pltpu.async_copy(h_hbm.at[src_v.at[SPW]],
                             rows_v.at[0], gsem.at[0]).wait()
            pltpu.async_copy(rows_v.at[0],
                             acc.at[dst_v.at[SPW]],
                             ssem.at[0], add=True).wait()

        plsc.subcore_barrier()

        @pl.when(s < NS - 1)
        def _():
            pltpu.sync_copy(acc.at[pl.ds(r0, RPT)],
                            out_hbm.at[c, pl.ds(r0, RPT)])

        @pl.when(s == NS - 1)
        def _():
            pltpu.sync_copy(acc.at[pl.ds(15 * RPT, RPT_LAST)],
                            out_hbm.at[c, pl.ds(15 * RPT, RPT_LAST)])

    return scatter


def _mm1_body(x_ref, w_ref, wb_ref, bb_ref, wl_ref, bl_ref,
              h_ref, mix_ref):
    relu = lambda v: jnp.maximum(v, 0.0)
    xd = x_ref[...]
    h_ref[...] = jnp.dot(xd, w_ref[...], preferred_element_type=jnp.float32)
    boxes = relu(jnp.dot(xd[:, 1:], wb_ref[...],
                         preferred_element_type=jnp.float32) + bb_ref[...])
    labels = relu(xd[:, 0:1] * wl_ref[...] + bl_ref[...])
    mix_ref[...] = boxes + labels


def _mm2_body(p_ref, w_ref, o_ref):
    x = p_ref[0] + p_ref[1]
    o_ref[...] = jnp.dot(x, w_ref[...], preferred_element_type=jnp.float32)


def _head_body(x0_ref, x1_ref, mixr_ref, cl_ref, wd1_ref, bd1_ref,
               wd2_ref, bd2_ref, wd3_ref, bd3_ref,
               wlat_ref, blat_ref, o_ref):
    relu = lambda v: jnp.maximum(v, 0.0)
    f32 = jnp.float32
    xr = x0_ref[...] + x1_ref[...]
    mix = relu(jnp.dot(mixr_ref[...], wd1_ref[...],
                       preferred_element_type=f32) + bd1_ref[...])
    x = relu(jnp.dot(cl_ref[...], wd2_ref[0:NUM_OBJ_CLS],
                     preferred_element_type=f32)
             + jnp.dot(xr, wd2_ref[NUM_OBJ_CLS:],
                       preferred_element_type=f32)
             + bd2_ref[...])
    x = x + mix
    x = relu(jnp.dot(x, wd3_ref[...], preferred_element_type=f32)
             + bd3_ref[...])
    x = relu(jnp.dot(x, wd3_ref[...], preferred_element_type=f32)
             + bd3_ref[...])
    o_ref[...] = relu(jnp.dot(x, wlat_ref[...], preferred_element_type=f32)
                      + blat_ref[...])


def kernel(E, X_data, class_labels, W1, W2, Wb, bb, Wl, bl,
           Wd1, bd1, Wd2, bd2, Wd3, bd3, Wlat, blat):
    f32 = jnp.float32
    E3 = E.reshape(2, NSUP, SR)
    zeros64 = jnp.zeros((N, 64), f32)
    zeros16 = jnp.zeros((N, 16), f32)

    h1, mixn = pl.pallas_call(
        _mm1_body,
        out_shape=(jax.ShapeDtypeStruct((N, 64), f32),
                   jax.ShapeDtypeStruct((N, 16), f32)),
    )(X_data, W1, Wb, bb.reshape(1, 16), Wl, bl.reshape(1, 16))

    p1 = _make_scatter(64)(E3, h1, zeros64)

    h2 = pl.pallas_call(
        _mm2_body,
        out_shape=jax.ShapeDtypeStruct((N, 16), f32),
    )(p1, W2)

    p2 = _make_scatter(16)(E3, h2, zeros16)

    x0_r = p2[0].reshape(BATCH, NUM_NODES * 16)
    x1_r = p2[1].reshape(BATCH, NUM_NODES * 16)
    mix_r = mixn.reshape(BATCH, NUM_NODES * 16)
    cl = class_labels.reshape(BATCH, -1)

    z = pl.pallas_call(
        _head_body,
        out_shape=jax.ShapeDtypeStruct((BATCH, 128), f32),
    )(x0_r, x1_r, mix_r, cl, Wd1, bd1.reshape(1, -1),
      Wd2, bd2.reshape(1, -1),
      Wd3, bd3.reshape(1, -1), Wlat, blat.reshape(1, -1))

    return (z, z)


# speedup vs baseline: 1.0656x; 1.0051x over previous
import functools

import jax
import jax.numpy as jnp
from jax import lax
from jax.experimental import pallas as pl
from jax.experimental.pallas import tpu as pltpu
from jax.experimental.pallas import tpu_sc as plsc

NUM_NODES = 100
BATCH = 100
N = NUM_NODES * BATCH
E_NUM = 320000
NUM_OBJ_CLS = 100
NC, NS = 2, 16
NW = NC * NS
CHUNK = 128
EROWS = E_NUM // CHUNK
RPW = EROWS // NW
XROWS = EROWS - RPW * NW
NBUF = 6
RPT = 624
RPT_LAST = N - 15 * RPT


@functools.cache
def _make_scatter(D):
    mesh = plsc.VectorSubcoreMesh(core_axis_name="c", subcore_axis_name="s",
                                  num_cores=NC, num_subcores=NS)

    @functools.partial(
        pl.kernel,
        out_type=jax.ShapeDtypeStruct((NC, N, D), jnp.float32),
        mesh=mesh,
        scratch_types=[
            pltpu.VMEM((RPW + 1, CHUNK), jnp.int32),
            pltpu.VMEM((RPW + 1, CHUNK), jnp.int32),
            pltpu.VMEM((NBUF, CHUNK, D), jnp.float32),
            pltpu.VMEM_SHARED((N, D), jnp.float32),
            pltpu.SemaphoreType.DMA((NBUF,)),
            pltpu.SemaphoreType.DMA((NBUF,)),
        ],
        compiler_params=pltpu.CompilerParams(use_tc_tiling_on_sc=False),
    )
    def scatter(e_hbm, h_hbm, zeros_hbm, out_hbm,
                src_v, dst_v, rows_v, acc, gsem, ssem):
        c = lax.axis_index("c")
        s = lax.axis_index("s")
        wid = s * NC + c
        r0 = pl.multiple_of(s * RPT, 8)

        row0 = wid * RPW
        pltpu.sync_copy(e_hbm.at[0, pl.ds(row0, RPW)], src_v.at[pl.ds(0, RPW)])
        pltpu.sync_copy(e_hbm.at[1, pl.ds(row0, RPW)], dst_v.at[pl.ds(0, RPW)])

        @pl.when(wid < XROWS)
        def _():
            xr = NW * RPW + wid
            pltpu.sync_copy(e_hbm.at[0, pl.ds(xr, 1)], src_v.at[pl.ds(RPW, 1)])
            pltpu.sync_copy(e_hbm.at[1, pl.ds(xr, 1)], dst_v.at[pl.ds(RPW, 1)])

        @pl.when(s < NS - 1)
        def _():
            pltpu.sync_copy(zeros_hbm.at[pl.ds(r0, RPT)],
                            acc.at[pl.ds(r0, RPT)])

        @pl.when(s == NS - 1)
        def _():
            pltpu.sync_copy(zeros_hbm.at[pl.ds(15 * RPT, RPT_LAST)],
                            acc.at[pl.ds(15 * RPT, RPT_LAST)])

        plsc.subcore_barrier()

        for b in range(NBUF):
            pltpu.async_copy(h_hbm.at[src_v.at[b]], rows_v.at[b], gsem.at[b])

        def body(g, carry):
            for b in range(NBUF):
                i = g * NBUF + b
                pltpu.make_async_copy(h_hbm.at[src_v.at[i]],
                                      rows_v.at[b], gsem.at[b]).wait()
                pltpu.async_copy(rows_v.at[b], acc.at[dst_v.at[i]],
                                 ssem.at[b], add=True)
                pltpu.make_async_copy(rows_v.at[b], acc.at[dst_v.at[i]],
                                      ssem.at[b]).wait()

                @pl.when(i + NBUF < RPW)
                def _():
                    pltpu.async_copy(h_hbm.at[src_v.at[i + NBUF]],
                                     rows_v.at[b], gsem.at[b])

            return carry

        lax.fori_loop(0, RPW // NBUF, body, 0)

        @pl.when(wid < XROWS)
        def _():
            pltpu.async_copy(h_hbm.at[src_v.at[RPW]], rows_v.at[0],
                             gsem.at[0]).wait()
            pltpu.async_copy(rows_v.at[0], acc.at[dst_v.at[RPW]],
                             ssem.at[0], add=True).wait()

        plsc.subcore_barrier()

        @pl.when(s < NS - 1)
        def _():
            pltpu.sync_copy(acc.at[pl.ds(r0, RPT)],
                            out_hbm.at[c, pl.ds(r0, RPT)])

        @pl.when(s == NS - 1)
        def _():
            pltpu.sync_copy(acc.at[pl.ds(15 * RPT, RPT_LAST)],
                            out_hbm.at[c, pl.ds(15 * RPT, RPT_LAST)])

    return scatter


def _mm1_body(x_ref, w_ref, wb_ref, bb_ref, wl_ref, bl_ref,
              h_ref, mix_ref):
    relu = lambda v: jnp.maximum(v, 0.0)
    xd = x_ref[...]
    h_ref[...] = jnp.dot(xd, w_ref[...], preferred_element_type=jnp.float32)
    boxes = relu(jnp.dot(xd[:, 1:], wb_ref[...],
                         preferred_element_type=jnp.float32) + bb_ref[...])
    labels = relu(xd[:, 0:1] * wl_ref[...] + bl_ref[...])
    mix_ref[...] = boxes + labels


def _mm2_body(p_ref, w_ref, o_ref):
    x = p_ref[0] + p_ref[1]
    o_ref[...] = jnp.dot(x, w_ref[...], preferred_element_type=jnp.float32)


def _head_body(x0_ref, x1_ref, mixr_ref, cl_ref, wd1_ref, bd1_ref,
               wd2_ref, bd2_ref, wd3_ref, bd3_ref,
               wlat_ref, blat_ref, o_ref):
    relu = lambda v: jnp.maximum(v, 0.0)
    f32 = jnp.float32
    xr = x0_ref[...] + x1_ref[...]
    mix = relu(jnp.dot(mixr_ref[...], wd1_ref[...],
                       preferred_element_type=f32) + bd1_ref[...])
    x = relu(jnp.dot(cl_ref[...], wd2_ref[0:NUM_OBJ_CLS],
                     preferred_element_type=f32)
             + jnp.dot(xr, wd2_ref[NUM_OBJ_CLS:],
                       preferred_element_type=f32)
             + bd2_ref[...])
    x = x + mix
    x = relu(jnp.dot(x, wd3_ref[...], preferred_element_type=f32)
             + bd3_ref[...])
    x = relu(jnp.dot(x, wd3_ref[...], preferred_element_type=f32)
             + bd3_ref[...])
    o_ref[...] = relu(jnp.dot(x, wlat_ref[...], preferred_element_type=f32)
                      + blat_ref[...])


def kernel(E, X_data, class_labels, W1, W2, Wb, bb, Wl, bl,
           Wd1, bd1, Wd2, bd2, Wd3, bd3, Wlat, blat):
    f32 = jnp.float32
    E3 = E.reshape(2, EROWS, CHUNK)
    zeros64 = jnp.zeros((N, 64), f32)
    zeros16 = jnp.zeros((N, 16), f32)

    h1, mixn = pl.pallas_call(
        _mm1_body,
        out_shape=(jax.ShapeDtypeStruct((N, 64), f32),
                   jax.ShapeDtypeStruct((N, 16), f32)),
    )(X_data, W1, Wb, bb.reshape(1, 16), Wl, bl.reshape(1, 16))

    p1 = _make_scatter(64)(E3, h1, zeros64)

    h2 = pl.pallas_call(
        _mm2_body,
        out_shape=jax.ShapeDtypeStruct((N, 16), f32),
    )(p1, W2)

    p2 = _make_scatter(16)(E3, h2, zeros16)

    x0_r = p2[0].reshape(BATCH, NUM_NODES * 16)
    x1_r = p2[1].reshape(BATCH, NUM_NODES * 16)
    mix_r = mixn.reshape(BATCH, NUM_NODES * 16)
    cl = class_labels.reshape(BATCH, -1)

    z = pl.pallas_call(
        _head_body,
        out_shape=jax.ShapeDtypeStruct((BATCH, 128), f32),
    )(x0_r, x1_r, mix_r, cl, Wd1, bd1.reshape(1, -1),
      Wd2, bd2.reshape(1, -1),
      Wd3, bd3.reshape(1, -1), Wlat, blat.reshape(1, -1))

    return (z, z)
